# baseline (speedup 1.0000x reference)
import jax
import jax.numpy as jnp
from jax.experimental import pallas as pl
from jax.experimental.pallas import tpu as pltpu

_NCH = 4


def _bigru_kernel(tok_ref, emb_ref, h0_ref, wih_ref, whh_ref, bih_ref,
                  bhh_ref, y_ref, hn_ref, x_s, gxf_ref, gxb_ref, sem):
    T, Bh, E = x_s.shape
    H = h0_ref.shape[-1]
    B = tok_ref.shape[0] // T
    Tc = T // _NCH

    p = pl.program_id(0)
    row0 = p * Bh

    def issue_chunk(c):
        for t in range(c * Tc, (c + 1) * Tc):
            for b in range(Bh):
                idx = tok_ref[(row0 + b) * T + t]
                pltpu.make_async_copy(
                    emb_ref.at[idx], x_s.at[t, b], sem.at[c]).start()

    def wait_chunk(c):
        pltpu.make_async_copy(
            x_s.at[pl.ds(c * Tc, Tc)], x_s.at[pl.ds(c * Tc, Tc)],
            sem.at[c]).wait()

    def issue_pair(j):
        issue_chunk(j)
        issue_chunk(_NCH - 1 - j)

    def wait_pair(j):
        wait_chunk(j)
        wait_chunk(_NCH - 1 - j)

    issue_pair(0)

    def fuse_w(ref, d):
        return jnp.concatenate(
            [ref[d, 0], ref[d, 1], ref[d, 2]], axis=-1).astype(jnp.bfloat16)

    def fuse_b(d):
        return jnp.concatenate(
            [bih_ref[d, 0] + bhh_ref[d, 0],
             bih_ref[d, 1] + bhh_ref[d, 1],
             bih_ref[d, 2]], axis=-1)

    wih_f = fuse_w(wih_ref, 0)
    wih_b = fuse_w(wih_ref, 1)
    whh_f = fuse_w(whh_ref, 0)
    whh_b = fuse_w(whh_ref, 1)
    bx_f = fuse_b(0)
    bx_b = fuse_b(1)
    bn_f = bhh_ref[0, 2]
    bn_b = bhh_ref[1, 2]

    def gx_chunk(c):
        x2d = x_s[pl.ds(c * Tc, Tc)].reshape(Tc * Bh, E).astype(jnp.bfloat16)
        gxf_ref[pl.ds(c * Tc, Tc)] = (
            jnp.dot(x2d, wih_f, preferred_element_type=jnp.float32)
            + bx_f).reshape(Tc, Bh, 3 * H)
        gxb_ref[pl.ds(c * Tc, Tc)] = (
            jnp.dot(x2d, wih_b, preferred_element_type=jnp.float32)
            + bx_b).reshape(Tc, Bh, 3 * H)

    def sig(u):
        return 0.5 + 0.5 * jnp.tanh(0.5 * u)

    def step(h, gx_s, whh, bn):
        gh = jnp.dot(h.astype(jnp.bfloat16), whh,
                     preferred_element_type=jnp.float32)
        r = sig(gx_s[:, :H] + gh[:, :H])
        z = sig(gx_s[:, H:2 * H] + gh[:, H:2 * H])
        n = jnp.tanh(gx_s[:, 2 * H:] + r * (gh[:, 2 * H:] + bn))
        return n + z * (h - n)

    hf = h0_ref[0]
    hb = h0_ref[1]

    for k in range(_NCH):
        if k < _NCH // 2:
            wait_pair(k)
            gx_chunk(k)
            gx_chunk(_NCH - 1 - k)
            if k + 1 < _NCH // 2:
                issue_pair(k + 1)
        cb = _NCH - 1 - k
        for tt in range(Tc):
            t = k * Tc + tt
            sb = cb * Tc + (Tc - 1 - tt)
            hf = step(hf, gxf_ref[t], whh_f, bn_f)
            hb = step(hb, gxb_ref[sb], whh_b, bn_b)
            y_ref[:, t, :H] = hf
            y_ref[:, sb, H:] = hb
    hn_ref[0] = hf
    hn_ref[1] = hb


def kernel(token_ids, h0, embedding, w_ih, w_hh, b_ih, b_hh):
    B, T = token_ids.shape
    E = embedding.shape[1]
    H = h0.shape[-1]
    Bh = B // 2

    output, hn = pl.pallas_call(
        _bigru_kernel,
        out_shape=(jax.ShapeDtypeStruct((B, T, 2 * H), jnp.float32),
                   jax.ShapeDtypeStruct((2, B, H), jnp.float32)),
        grid_spec=pltpu.PrefetchScalarGridSpec(
            num_scalar_prefetch=1,
            grid=(2,),
            in_specs=[
                pl.BlockSpec(memory_space=pl.ANY),
                pl.BlockSpec((2, Bh, H), lambda p, tok: (0, p, 0)),
                pl.BlockSpec((2, 3, E, H), lambda p, tok: (0, 0, 0, 0)),
                pl.BlockSpec((2, 3, H, H), lambda p, tok: (0, 0, 0, 0)),
                pl.BlockSpec((2, 3, 1, H), lambda p, tok: (0, 0, 0, 0)),
                pl.BlockSpec((2, 3, 1, H), lambda p, tok: (0, 0, 0, 0)),
            ],
            out_specs=(pl.BlockSpec((Bh, T, 2 * H), lambda p, tok: (p, 0, 0)),
                       pl.BlockSpec((2, Bh, H), lambda p, tok: (0, p, 0))),
            scratch_shapes=[
                pltpu.VMEM((T, Bh, E), jnp.float32),
                pltpu.VMEM((T, Bh, 3 * H), jnp.float32),
                pltpu.VMEM((T, Bh, 3 * H), jnp.float32),
                pltpu.SemaphoreType.DMA((_NCH,)),
            ],
        ),
        compiler_params=pltpu.CompilerParams(
            dimension_semantics=("parallel",),
            disable_bounds_checks=True),
    )(token_ids.reshape(B * T), embedding, h0, w_ih, w_hh, b_ih, b_hh)

    return output, hn


# speedup vs baseline: 1.0376x; 1.0376x over previous
import jax
import jax.numpy as jnp
from jax.experimental import pallas as pl
from jax.experimental.pallas import tpu as pltpu

_NCH = 4


def _bigru_kernel(tok_ref, emb_ref, h0_ref, wih_ref, whh_ref, bih_ref,
                  bhh_ref, y_ref, hn_ref, x_s, gxf_ref, gxb_ref, sem):
    T, Bh, E = x_s.shape
    H = h0_ref.shape[-1]
    B = tok_ref.shape[0] // T
    Tc = T // _NCH

    p = pl.program_id(0)
    row0 = p * Bh

    def issue_chunk(c):
        for t in range(c * Tc, (c + 1) * Tc):
            for b in range(Bh):
                idx = tok_ref[(row0 + b) * T + t]
                pltpu.make_async_copy(
                    emb_ref.at[idx], x_s.at[t, b], sem.at[c]).start()

    def wait_chunk(c):
        pltpu.make_async_copy(
            x_s.at[pl.ds(c * Tc, Tc)], x_s.at[pl.ds(c * Tc, Tc)],
            sem.at[c]).wait()

    def issue_pair(j):
        issue_chunk(j)
        issue_chunk(_NCH - 1 - j)

    def wait_pair(j):
        wait_chunk(j)
        wait_chunk(_NCH - 1 - j)

    issue_pair(0)

    def fuse_w(ref, d):
        return jnp.concatenate(
            [ref[d, 0], ref[d, 1], ref[d, 2]], axis=-1).astype(jnp.bfloat16)

    def fuse_b(d):
        return jnp.concatenate(
            [bih_ref[d, 0] + bhh_ref[d, 0],
             bih_ref[d, 1] + bhh_ref[d, 1],
             bih_ref[d, 2]], axis=-1)

    wih_f = fuse_w(wih_ref, 0)
    wih_b = fuse_w(wih_ref, 1)
    whh_f = fuse_w(whh_ref, 0)
    whh_b = fuse_w(whh_ref, 1)
    bx_f = fuse_b(0)
    bx_b = fuse_b(1)
    bn_f = bhh_ref[0, 2]
    bn_b = bhh_ref[1, 2]

    def gx_chunk(c):
        x2d = x_s[pl.ds(c * Tc, Tc)].reshape(Tc * Bh, E).astype(jnp.bfloat16)
        gxf_ref[pl.ds(c * Tc, Tc)] = (
            jnp.dot(x2d, wih_f, preferred_element_type=jnp.float32)
            + bx_f).reshape(Tc, Bh, 3 * H)
        gxb_ref[pl.ds(c * Tc, Tc)] = (
            jnp.dot(x2d, wih_b, preferred_element_type=jnp.float32)
            + bx_b).reshape(Tc, Bh, 3 * H)

    def sig(u):
        return 0.5 + 0.5 * jnp.tanh(0.5 * u)

    def step(h, gx_s, whh, bn):
        gh = jnp.dot(h.astype(jnp.bfloat16), whh,
                     preferred_element_type=jnp.float32)
        r = sig(gx_s[:, :H] + gh[:, :H])
        z = sig(gx_s[:, H:2 * H] + gh[:, H:2 * H])
        n = jnp.tanh(gx_s[:, 2 * H:] + r * (gh[:, 2 * H:] + bn))
        return n + z * (h - n)

    hf = h0_ref[0]
    hb = h0_ref[1]

    for k in range(_NCH):
        if k < _NCH // 2:
            wait_pair(k)
            if k + 1 < _NCH // 2:
                issue_pair(k + 1)
            gx_chunk(k)
            gx_chunk(_NCH - 1 - k)
        cb = _NCH - 1 - k
        for tt in range(Tc):
            t = k * Tc + tt
            sb = cb * Tc + (Tc - 1 - tt)
            hf = step(hf, gxf_ref[t], whh_f, bn_f)
            hb = step(hb, gxb_ref[sb], whh_b, bn_b)
            y_ref[:, t, :H] = hf
            y_ref[:, sb, H:] = hb
    hn_ref[0] = hf
    hn_ref[1] = hb


def kernel(token_ids, h0, embedding, w_ih, w_hh, b_ih, b_hh):
    B, T = token_ids.shape
    E = embedding.shape[1]
    H = h0.shape[-1]
    Bh = B // 2

    output, hn = pl.pallas_call(
        _bigru_kernel,
        out_shape=(jax.ShapeDtypeStruct((B, T, 2 * H), jnp.float32),
                   jax.ShapeDtypeStruct((2, B, H), jnp.float32)),
        grid_spec=pltpu.PrefetchScalarGridSpec(
            num_scalar_prefetch=1,
            grid=(2,),
            in_specs=[
                pl.BlockSpec(memory_space=pl.ANY),
                pl.BlockSpec((2, Bh, H), lambda p, tok: (0, p, 0)),
                pl.BlockSpec((2, 3, E, H), lambda p, tok: (0, 0, 0, 0)),
                pl.BlockSpec((2, 3, H, H), lambda p, tok: (0, 0, 0, 0)),
                pl.BlockSpec((2, 3, 1, H), lambda p, tok: (0, 0, 0, 0)),
                pl.BlockSpec((2, 3, 1, H), lambda p, tok: (0, 0, 0, 0)),
            ],
            out_specs=(pl.BlockSpec((Bh, T, 2 * H), lambda p, tok: (p, 0, 0)),
                       pl.BlockSpec((2, Bh, H), lambda p, tok: (0, p, 0))),
            scratch_shapes=[
                pltpu.VMEM((T, Bh, E), jnp.float32),
                pltpu.VMEM((T, Bh, 3 * H), jnp.float32),
                pltpu.VMEM((T, Bh, 3 * H), jnp.float32),
                pltpu.SemaphoreType.DMA((_NCH,)),
            ],
        ),
        compiler_params=pltpu.CompilerParams(
            dimension_semantics=("parallel",),
            disable_bounds_checks=True),
    )(token_ids.reshape(B * T), embedding, h0, w_ih, w_hh, b_ih, b_hh)

    return output, hn
